# Initial kernel scaffold; baseline (speedup 1.0000x reference)
import jax
import jax.numpy as jnp
from jax.experimental import pallas as pl


def kernel(feature, edge_index, W1, a1_src, a1_dst, W2, a2_src, a2_dst, W3, a3_src, a3_dst):
    raise NotImplementedError("write your pallas kernel here")


# speedup vs baseline: 26.2680x; 26.2680x over previous
import functools

import jax
import jax.numpy as jnp
from jax import lax
from jax.experimental import pallas as pl
from jax.experimental.pallas import tpu as pltpu
from jax.experimental.pallas import tpu_sc as plsc

N = 10000
E = 320000
D_IN = 128
H1 = 64
H2 = 32

NC = 2
NS = 16
NW = NC * NS

CHUNK = 128
NCHUNKS = -(-E // CHUNK)
CW = -(-NCHUNKS // NW)
EPAD = CW * NW * CHUNK
NPAD = 10112


def _tc_first(x_ref, w_ref, a_ref, h_ref, sd_ref):
    h = jnp.dot(x_ref[...], w_ref[...], preferred_element_type=jnp.float32)
    h_ref[...] = h
    sd_ref[...] = jnp.dot(h, a_ref[...], preferred_element_type=jnp.float32)


def _tc_mid(acc_ref, den_ref, w_ref, a_ref, h_ref, sd_ref):
    acc = acc_ref[0, :N, :] + acc_ref[1, :N, :]
    den = den_ref[0, :N, :] + den_ref[1, :N, :]
    x = jnp.maximum(acc / (den + 1e-16), 0.0)
    h = jnp.dot(x, w_ref[...], preferred_element_type=jnp.float32)
    h_ref[...] = h
    sd_ref[...] = jnp.dot(h, a_ref[...], preferred_element_type=jnp.float32)


def _tc_final(num_ref, den_ref, out_ref):
    num = num_ref[0, :N, :] + num_ref[1, :N, :]
    den = den_ref[0, :N, :] + den_ref[1, :N, :]
    out_ref[...] = jax.nn.sigmoid(num / (den + 1e-16))


def _make_sc_edge_pass(F):
    mesh = plsc.VectorSubcoreMesh(core_axis_name="c", subcore_axis_name="s")
    sd_len = 2 * N + 32
    rslice = NPAD // NS

    @functools.partial(
        pl.kernel,
        out_type=[
            jax.ShapeDtypeStruct((NC, NPAD, F), jnp.float32),
            jax.ShapeDtypeStruct((NC * NPAD,), jnp.float32),
        ],
        mesh=mesh,
        compiler_params=pltpu.CompilerParams(
            needs_layout_passes=False, use_tc_tiling_on_sc=False),
        scratch_types=[
            pltpu.VMEM((sd_len,), jnp.float32),
            pltpu.VMEM((CHUNK,), jnp.int32),
            pltpu.VMEM((CHUNK,), jnp.int32),
            pltpu.VMEM((CHUNK,), jnp.float32),
            pltpu.VMEM((CHUNK, F), jnp.float32),
            pltpu.VMEM_SHARED((NPAD, F), jnp.float32),
            pltpu.VMEM_SHARED((NPAD,), jnp.float32),
            pltpu.SemaphoreType.DMA,
        ],
    )
    def edge_pass(src_hbm, dst_hbm, sd_hbm, h_hbm,
                  acc_out, den_out,
                  sd_tab, src_buf, dst_buf, w_buf, rows_buf,
                  acc_sh, den_sh, sem):
        c = lax.axis_index("c")
        s = lax.axis_index("s")
        w = s * NC + c

        pltpu.sync_copy(sd_hbm, sd_tab)
        zero16 = jnp.zeros((16,), jnp.float32)
        for ed in range(CHUNK):
            for t in range(F // 16):
                rows_buf[ed, pl.ds(t * 16, 16)] = zero16
        for j in range(CHUNK // 16):
            w_buf[pl.ds(j * 16, 16)] = zero16
        r0 = s * rslice
        nfull, rem = divmod(rslice, CHUNK)
        for k in range(nfull + (1 if rem else 0)):
            ln = CHUNK if k < nfull else rem
            off = r0 + k * CHUNK
            pltpu.sync_copy(rows_buf.at[pl.ds(0, ln)], acc_sh.at[pl.ds(off, ln)])
            pltpu.sync_copy(w_buf.at[pl.ds(0, ln)], den_sh.at[pl.ds(off, ln)])
        plsc.subcore_barrier()

        def chunk_body(i, carry):
            base = (w * CW + i) * CHUNK
            pltpu.sync_copy(src_hbm.at[pl.ds(base, CHUNK)], src_buf)
            pltpu.sync_copy(dst_hbm.at[pl.ds(base, CHUNK)], dst_buf)
            gcp = pltpu.async_copy(h_hbm.at[src_buf], rows_buf, sem)
            for j in range(CHUNK // 16):
                s16 = src_buf[pl.ds(j * 16, 16)]
                d16 = dst_buf[pl.ds(j * 16, 16)]
                sv = plsc.load_gather(sd_tab, [s16 * 2])
                dv = plsc.load_gather(sd_tab, [d16 * 2 + 1])
                e = sv + dv
                e = jnp.where(e >= 0.0, e, e * 0.2)
                w_buf[pl.ds(j * 16, 16)] = jnp.exp(e)
            gcp.wait()
            for ed in range(CHUNK):
                ws = plsc.load_gather(w_buf, [jnp.full((16,), ed, jnp.int32)])
                for t in range(F // 16):
                    rows_buf[ed, pl.ds(t * 16, 16)] = (
                        rows_buf[ed, pl.ds(t * 16, 16)] * ws)
            pltpu.sync_copy(rows_buf, acc_sh.at[dst_buf], add=True)
            pltpu.sync_copy(w_buf, den_sh.at[dst_buf], add=True)
            return carry

        lax.fori_loop(0, CW, chunk_body, 0)
        plsc.subcore_barrier()
        for k in range(nfull + (1 if rem else 0)):
            ln = CHUNK if k < nfull else rem
            off = r0 + k * CHUNK
            pltpu.sync_copy(acc_sh.at[pl.ds(off, ln)], rows_buf.at[pl.ds(0, ln)])
            pltpu.sync_copy(rows_buf.at[pl.ds(0, ln)], acc_out.at[c, pl.ds(off, ln)])
            pltpu.sync_copy(den_sh.at[pl.ds(off, ln)], w_buf.at[pl.ds(0, ln)])
            pltpu.sync_copy(w_buf.at[pl.ds(0, ln)], den_out.at[pl.ds(c * NPAD + off, ln)])

    return edge_pass


def _make_sc_edge_pass_scalar():
    mesh = plsc.VectorSubcoreMesh(core_axis_name="c", subcore_axis_name="s")
    sd_len = 2 * N + 32
    h_len = N + 16
    rslice = NPAD // NS

    @functools.partial(
        pl.kernel,
        out_type=[
            jax.ShapeDtypeStruct((NC * NPAD,), jnp.float32),
            jax.ShapeDtypeStruct((NC * NPAD,), jnp.float32),
        ],
        mesh=mesh,
        compiler_params=pltpu.CompilerParams(
            needs_layout_passes=False, use_tc_tiling_on_sc=False),
        scratch_types=[
            pltpu.VMEM((sd_len,), jnp.float32),
            pltpu.VMEM((h_len,), jnp.float32),
            pltpu.VMEM((CHUNK,), jnp.int32),
            pltpu.VMEM((CHUNK,), jnp.int32),
            pltpu.VMEM((CHUNK,), jnp.float32),
            pltpu.VMEM((CHUNK,), jnp.float32),
            pltpu.VMEM_SHARED((NPAD,), jnp.float32),
            pltpu.VMEM_SHARED((NPAD,), jnp.float32),
        ],
    )
    def edge_pass(src_hbm, dst_hbm, sd_hbm, h_hbm,
                  num_out, den_out,
                  sd_tab, h_tab, src_buf, dst_buf, w_buf, num_buf,
                  num_sh, den_sh):
        c = lax.axis_index("c")
        s = lax.axis_index("s")
        w = s * NC + c

        pltpu.sync_copy(sd_hbm, sd_tab)
        pltpu.sync_copy(h_hbm, h_tab)
        zero16 = jnp.zeros((16,), jnp.float32)
        for j in range(CHUNK // 16):
            w_buf[pl.ds(j * 16, 16)] = zero16
        r0 = s * rslice
        nfull, rem = divmod(rslice, CHUNK)
        for k in range(nfull + (1 if rem else 0)):
            ln = CHUNK if k < nfull else rem
            off = r0 + k * CHUNK
            pltpu.sync_copy(w_buf.at[pl.ds(0, ln)], num_sh.at[pl.ds(off, ln)])
            pltpu.sync_copy(w_buf.at[pl.ds(0, ln)], den_sh.at[pl.ds(off, ln)])
        plsc.subcore_barrier()

        def chunk_body(i, carry):
            base = (w * CW + i) * CHUNK
            pltpu.sync_copy(src_hbm.at[pl.ds(base, CHUNK)], src_buf)
            pltpu.sync_copy(dst_hbm.at[pl.ds(base, CHUNK)], dst_buf)
            for j in range(CHUNK // 16):
                s16 = src_buf[pl.ds(j * 16, 16)]
                d16 = dst_buf[pl.ds(j * 16, 16)]
                sv = plsc.load_gather(sd_tab, [s16 * 2])
                dv = plsc.load_gather(sd_tab, [d16 * 2 + 1])
                e = sv + dv
                e = jnp.where(e >= 0.0, e, e * 0.2)
                wv = jnp.exp(e)
                hv = plsc.load_gather(h_tab, [s16])
                w_buf[pl.ds(j * 16, 16)] = wv
                num_buf[pl.ds(j * 16, 16)] = wv * hv
            pltpu.sync_copy(num_buf, num_sh.at[dst_buf], add=True)
            pltpu.sync_copy(w_buf, den_sh.at[dst_buf], add=True)
            return carry

        lax.fori_loop(0, CW, chunk_body, 0)
        plsc.subcore_barrier()
        for k in range(nfull + (1 if rem else 0)):
            ln = CHUNK if k < nfull else rem
            off = r0 + k * CHUNK
            pltpu.sync_copy(num_sh.at[pl.ds(off, ln)], num_buf.at[pl.ds(0, ln)])
            pltpu.sync_copy(num_buf.at[pl.ds(0, ln)], num_out.at[pl.ds(c * NPAD + off, ln)])
            pltpu.sync_copy(den_sh.at[pl.ds(off, ln)], w_buf.at[pl.ds(0, ln)])
            pltpu.sync_copy(w_buf.at[pl.ds(0, ln)], den_out.at[pl.ds(c * NPAD + off, ln)])

    return edge_pass


_sc_pass_64 = _make_sc_edge_pass(H1)
_sc_pass_32 = _make_sc_edge_pass(H2)
_sc_pass_1 = _make_sc_edge_pass_scalar()


def kernel(feature, edge_index, W1, a1_src, a1_dst, W2, a2_src, a2_dst,
           W3, a3_src, a3_dst):
    f32 = jnp.float32
    src = edge_index[0]
    dst = edge_index[1]
    pad_e = EPAD - E
    src_p = jnp.concatenate([src, jnp.zeros((pad_e,), jnp.int32)])
    dst_p = jnp.concatenate([dst, jnp.full((pad_e,), N, jnp.int32)])

    A1 = jnp.stack([a1_src, a1_dst], axis=1)
    A2 = jnp.stack([a2_src, a2_dst], axis=1)
    A3 = jnp.stack([a3_src, a3_dst], axis=1)

    sd_pad = jnp.zeros((32,), f32)
    h_pad = jnp.zeros((16,), f32)

    h1, sd1 = pl.pallas_call(
        _tc_first,
        out_shape=[jax.ShapeDtypeStruct((N, H1), f32),
                   jax.ShapeDtypeStruct((N, 2), f32)],
    )(feature, W1, A1)

    acc1, den1 = _sc_pass_64(
        src_p, dst_p, jnp.concatenate([sd1.reshape(-1), sd_pad]), h1)

    h2, sd2 = pl.pallas_call(
        _tc_mid,
        out_shape=[jax.ShapeDtypeStruct((N, H2), f32),
                   jax.ShapeDtypeStruct((N, 2), f32)],
    )(acc1, den1.reshape(NC, NPAD, 1), W2, A2)

    acc2, den2 = _sc_pass_32(
        src_p, dst_p, jnp.concatenate([sd2.reshape(-1), sd_pad]), h2)

    h3, sd3 = pl.pallas_call(
        _tc_mid,
        out_shape=[jax.ShapeDtypeStruct((N, 1), f32),
                   jax.ShapeDtypeStruct((N, 2), f32)],
    )(acc2, den2.reshape(NC, NPAD, 1), W3, A3)

    num3, den3 = _sc_pass_1(
        src_p, dst_p, jnp.concatenate([sd3.reshape(-1), sd_pad]),
        jnp.concatenate([h3.reshape(-1), h_pad]))

    out = pl.pallas_call(
        _tc_final,
        out_shape=jax.ShapeDtypeStruct((N, 1), f32),
    )(num3.reshape(NC, NPAD, 1), den3.reshape(NC, NPAD, 1))
    return out
